# baseline (speedup 1.0000x reference)
import functools

import jax
import jax.numpy as jnp
from jax import lax
from jax.experimental import pallas as pl
from jax.experimental.pallas import tpu as pltpu
from jax.experimental.pallas import tpu_sc as plsc

N = 10000
E = 320000
IN_DIM = 128
H = 32
T = 4
O = 5

NC = 2
NS = 16
NW = NC * NS
NP = 10240
TPN = NP // NS
EW = E // NW
C = 1000
NCH = EW // C

_MESH = plsc.VectorSubcoreMesh(
    core_axis_name="c", subcore_axis_name="s", num_cores=NC, num_subcores=NS)


def _elu(v):
    return jnp.where(v > 0, v, jnp.exp(jnp.minimum(v, 0.0)) - 1.0)


def _ln(v, g, b):
    mu = jnp.mean(v, axis=-1, keepdims=True)
    var = jnp.var(v, axis=-1, keepdims=True)
    return (v - mu) / jnp.sqrt(var + 1e-5) * g + b


def _zero_rows(ref, nrows, width):
    zero = jnp.zeros((16,), jnp.float32)

    def body(i, carry):
        for w0 in range(0, width, 16):
            ref[i, pl.ds(w0, 16)] = zero
        return carry

    lax.fori_loop(0, nrows, body, 0)


@functools.partial(
    pl.kernel,
    out_type=jax.ShapeDtypeStruct((NC, NP, 16), jnp.float32),
    mesh=_MESH,
    compiler_params=pltpu.CompilerParams(use_tc_tiling_on_sc=False, needs_layout_passes=False),
    scratch_types=[
        pltpu.VMEM((C,), jnp.int32),
        pltpu.VMEM((C, 16), jnp.float32),
        pltpu.VMEM_SHARED((NP, 16), jnp.float32),
    ],
)
def _sc_deg(ei_hbm, out_hbm, didx, ones, acc):
    cid = lax.axis_index("c")
    sid = lax.axis_index("s")
    wid = sid * NC + cid

    _zero_rows(ones, TPN, 16)
    pltpu.sync_copy(ones.at[pl.ds(0, TPN)], acc.at[pl.ds(sid * TPN, TPN)])

    one = jnp.ones((16,), jnp.float32)

    def fill(i, carry):
        ones[i, :] = one
        return carry

    lax.fori_loop(0, C, fill, 0)
    plsc.subcore_barrier()

    def chunk(i, carry):
        b = pl.multiple_of(wid * EW + i * C, 8)
        pltpu.sync_copy(ei_hbm.at[1, pl.ds(b, C)], didx)
        pltpu.sync_copy(ones, acc.at[didx], add=True)
        return carry

    lax.fori_loop(0, NCH, chunk, 0)
    plsc.subcore_barrier()
    pltpu.sync_copy(acc.at[pl.ds(sid * TPN, TPN)],
                    out_hbm.at[cid, pl.ds(sid * TPN, TPN)])


@functools.partial(
    pl.kernel,
    out_type=jax.ShapeDtypeStruct((NC, NP, H), jnp.float32),
    mesh=_MESH,
    compiler_params=pltpu.CompilerParams(use_tc_tiling_on_sc=False, needs_layout_passes=False),
    scratch_types=[
        pltpu.VMEM((C,), jnp.int32),
        pltpu.VMEM((C,), jnp.int32),
        pltpu.VMEM((C,), jnp.int32),
        pltpu.VMEM((C,), jnp.int32),
        pltpu.VMEM((C, H), jnp.float32),
        pltpu.VMEM((C, H), jnp.float32),
        pltpu.VMEM_SHARED((NP, H), jnp.float32),
        pltpu.SemaphoreType.DMA,
        pltpu.SemaphoreType.DMA,
    ],
)
def _sc_gcn_edges(xws_hbm, ei_hbm, out_hbm,
                  sidx0, sidx1, didx0, didx1, rows0, rows1, acc,
                  sem0, sem1):
    cid = lax.axis_index("c")
    sid = lax.axis_index("s")
    wid = sid * NC + cid

    _zero_rows(rows0, TPN, H)
    pltpu.sync_copy(rows0.at[pl.ds(0, TPN)], acc.at[pl.ds(sid * TPN, TPN)])
    plsc.subcore_barrier()

    sidx = (sidx0, sidx1)
    didx = (didx0, didx1)
    rows = (rows0, rows1)
    sem = (sem0, sem1)

    def issue(c, b):
        eb = pl.multiple_of(wid * EW + c * C, 8)
        pltpu.sync_copy(ei_hbm.at[0, pl.ds(eb, C)], sidx[b])
        pltpu.sync_copy(ei_hbm.at[1, pl.ds(eb, C)], didx[b])
        return pltpu.async_copy(xws_hbm.at[sidx[b]], rows[b], sem[b])

    cp = [None, None]
    cp[0] = issue(0, 0)
    for c in range(NCH):
        b = c % 2
        nb = (c + 1) % 2
        if c + 1 < NCH:
            cp[nb] = issue(c + 1, nb)
        cp[b].wait()
        pltpu.sync_copy(rows[b], acc.at[didx[b]], add=True)
    plsc.subcore_barrier()
    pltpu.sync_copy(acc.at[pl.ds(sid * TPN, TPN)],
                    out_hbm.at[cid, pl.ds(sid * TPN, TPN)])


PW = 48
CE = 400
NCHE = EW // CE
GPC = CE // 16


@functools.partial(
    pl.kernel,
    out_type=jax.ShapeDtypeStruct((NC, NP, PW), jnp.float32),
    mesh=_MESH,
    compiler_params=pltpu.CompilerParams(use_tc_tiling_on_sc=False, needs_layout_passes=False),
    scratch_types=[
        pltpu.VMEM((CE,), jnp.int32),
        pltpu.VMEM((CE,), jnp.int32),
        pltpu.VMEM((CE, H), jnp.float32),
        pltpu.VMEM((CE, H), jnp.float32),
        pltpu.VMEM((CE, 16), jnp.float32),
        pltpu.VMEM((CE, 16), jnp.float32),
        pltpu.VMEM((CE, PW), jnp.float32),
        pltpu.VMEM((48,), jnp.float32),
        pltpu.VMEM_SHARED((NP, PW), jnp.float32),
        pltpu.SemaphoreType.DMA,
        pltpu.SemaphoreType.DMA,
        pltpu.SemaphoreType.DMA,
        pltpu.SemaphoreType.DMA,
    ],
)
def _sc_exposure(h_hbm, a_hbm, b_hbm, econst_hbm, ei_hbm, out_hbm,
                 sidx, didx, hs, hd, ad, bs,
                 outb, econ, acc, sem_s, sem_d, sem_a, sem_b):
    cid = lax.axis_index("c")
    sid = lax.axis_index("s")
    wid = sid * NC + cid

    _zero_rows(outb, 320, PW)
    pltpu.sync_copy(outb.at[pl.ds(0, 320)], acc.at[pl.ds(sid * TPN, 320)])
    pltpu.sync_copy(outb.at[pl.ds(0, 320)],
                    acc.at[pl.ds(sid * TPN + 320, 320)])
    pltpu.sync_copy(econst_hbm, econ)
    plsc.subcore_barrier()

    w1v = econ[pl.ds(0, 16)]
    w2v = econ[pl.ds(16, 16)]
    b2v = econ[pl.ds(32, 16)]

    iota16 = lax.broadcasted_iota(jnp.int32, (16,), 0)
    m0 = jnp.where(iota16 == 0, 1.0, 0.0).astype(jnp.float32)

    _gdn = lax.GatherDimensionNumbers(
        offset_dims=(), collapsed_slice_dims=(0,), start_index_map=(0,))

    def vsum16(v):
        for sh in (8, 4, 2, 1):
            perm = jnp.bitwise_xor(iota16, sh)
            v = v + lax.gather(v, perm[:, None], dimension_numbers=_gdn,
                               slice_sizes=(1,),
                               mode=lax.GatherScatterMode.PROMISE_IN_BOUNDS)
        return v

    def chunk(i, carry):
        eb = pl.multiple_of(wid * EW + i * CE, 8)
        pltpu.sync_copy(ei_hbm.at[0, pl.ds(eb, CE)], sidx)
        pltpu.sync_copy(ei_hbm.at[1, pl.ds(eb, CE)], didx)
        cp_s = pltpu.async_copy(h_hbm.at[sidx], hs, sem_s)
        cp_d = pltpu.async_copy(h_hbm.at[didx], hd, sem_d)
        cp_a = pltpu.async_copy(a_hbm.at[didx], ad, sem_a)
        cp_b = pltpu.async_copy(b_hbm.at[sidx], bs, sem_b)
        cp_s.wait()
        cp_d.wait()
        cp_a.wait()
        cp_b.wait()

        def edge(r4, carry2):
            for j in range(4):
                r = r4 * 4 + j
                hs0 = hs[r, pl.ds(0, 16)]
                hs1 = hs[r, pl.ds(16, 16)]
                d0 = hd[r, pl.ds(0, 16)] - hs0
                d1 = hd[r, pl.ds(16, 16)] - hs1
                sv = vsum16(d0 * d0 + d1 * d1)
                sim = jnp.exp(sv * (-1.0 / (2.0 * H)))
                z = ad[r, :] + bs[r, :] + sim * w1v
                u = jnp.where(z > 0, z, jnp.exp(jnp.minimum(z, 0.0)) - 1.0)
                tv = vsum16(u * w2v) + b2v
                wgt = 1.0 / (1.0 + jnp.exp(-tv))
                outb[r, pl.ds(0, 16)] = hs0 * wgt
                outb[r, pl.ds(16, 16)] = hs1 * wgt
                outb[r, pl.ds(32, 16)] = wgt * m0
            return carry2

        lax.fori_loop(0, CE // 4, edge, 0)
        pltpu.sync_copy(outb, acc.at[didx], add=True)
        return carry

    lax.fori_loop(0, NCHE, chunk, 0)
    plsc.subcore_barrier()
    pltpu.sync_copy(acc.at[pl.ds(sid * TPN, TPN)],
                    out_hbm.at[cid, pl.ds(sid * TPN, TPN)])


RB = 2000
GRID_N = N // RB
EB = 8000
GRID_E = E // EB


def _full(shape):
    return pl.BlockSpec(shape, lambda i: tuple(0 for _ in shape))


def _rows(width):
    return pl.BlockSpec((RB, width), lambda i: (i, 0))


def _dot(a, b):
    return jnp.dot(a, b, preferred_element_type=jnp.float32)


def _tc_a_body(x, degp, ego_W1, ego_b1, ego_W2, ego_b2, g1_W,
               h_ego_o, xw1s_o, dinv_o):
    xb = x[...]
    dp = degp[...]
    h_ego_o[...] = _dot(_elu(_dot(xb, ego_W1[...]) + ego_b1[...]),
                        ego_W2[...]) + ego_b2[...]
    deg = 1.0 + dp[:, 0:1] + dp[:, 1:2]
    dinv = lax.rsqrt(deg)
    dinv_o[...] = dinv
    xw1s_o[...] = _dot(xb, g1_W[...]) * dinv


def _tc_a(x, degp, ego_W1, ego_b1, ego_W2, ego_b2, g1_W):
    return pl.pallas_call(
        _tc_a_body,
        grid=(GRID_N,),
        in_specs=[
            _rows(IN_DIM), _rows(2),
            _full((IN_DIM, H)), _full((H,)), _full((H, H)), _full((H,)),
            _full((IN_DIM, H)),
        ],
        out_specs=[_rows(H), _rows(H), _rows(1)],
        out_shape=[
            jax.ShapeDtypeStruct((N, H), jnp.float32),
            jax.ShapeDtypeStruct((N, H), jnp.float32),
            jax.ShapeDtypeStruct((N, 1), jnp.float32),
        ],
    )(x, degp, ego_W1, ego_b1, ego_W2, ego_b2, g1_W)


def _tc_b_body(a0, a1, xws, dinv, gb, lng, lnb, W2, xw2s_o):
    dv = dinv[...]
    g1out = dv * (a0[...] + a1[...] + xws[...]) + gb[...]
    h1 = _elu(_ln(g1out, lng[...], lnb[...]))
    xw2s_o[...] = _dot(h1, W2[...]) * dv


def _tc_b(a0, a1, xws, dinv, gb, lng, lnb, W2):
    return pl.pallas_call(
        _tc_b_body,
        grid=(GRID_N,),
        in_specs=[
            _rows(H), _rows(H), _rows(H), _rows(1),
            _full((H,)), _full((H,)), _full((H,)), _full((H, H)),
        ],
        out_specs=[_rows(H)],
        out_shape=[jax.ShapeDtypeStruct((N, H), jnp.float32)],
    )(a0, a1, xws, dinv, gb, lng, lnb, W2)


def _tc_c_body(a0, a1, xws, dinv, gb, lng, lnb, eW1, eb1, h_o, a_o, b_o):
    g2out = dinv[...] * (a0[...] + a1[...] + xws[...]) + gb[...]
    h = _elu(_ln(g2out, lng[...], lnb[...]))
    h_o[...] = h
    w1 = eW1[...]
    a_o[...] = _dot(h, w1[0:H]) + eb1[...]
    b_o[...] = _dot(h, w1[H:2 * H])


def _tc_c(a0, a1, xws, dinv, gb, lng, lnb, eW1, eb1):
    return pl.pallas_call(
        _tc_c_body,
        grid=(GRID_N,),
        in_specs=[
            _rows(H), _rows(H), _rows(H), _rows(1),
            _full((H,)), _full((H,)), _full((H,)),
            _full((2 * H + 1, 16)), _full((16,)),
        ],
        out_specs=[_rows(H), _rows(16), _rows(16)],
        out_shape=[
            jax.ShapeDtypeStruct((N, H), jnp.float32),
            jax.ShapeDtypeStruct((N, 16), jnp.float32),
            jax.ShapeDtypeStruct((N, 16), jnp.float32),
        ],
    )(a0, a1, xws, dinv, gb, lng, lnb, eW1, eb1)


def _tc_e_body(h_ego, h, ae0, ae1,
               out_W1, out_b1, out_W2, out_b2, out_W3, out_b3,
               loc_W1, loc_b1, loc_W2, loc_b2,
               mu_W1, mu_b1, mu_W2, mu_b2,
               lv_W1, lv_b1, lv_W2, lv_b2,
               yf_o, yl_o, mu_o, lv_o):
    he = h_ego[...]
    hb = h[...]
    a = ae0[...] + ae1[...]
    h_exp = a[:, 0:H] / jnp.maximum(a[:, H:H + 1], 1e-8)
    h_full = jnp.concatenate([he, hb, h_exp], axis=-1)

    def softmax(v):
        m = jnp.max(v, axis=-1, keepdims=True)
        e = jnp.exp(v - m)
        return e / jnp.sum(e, axis=-1, keepdims=True)

    o = _elu(_dot(h_full, out_W1[...]) + out_b1[...])
    o = _elu(_dot(o, out_W2[...]) + out_b2[...])
    yf_o[...] = softmax(_dot(o, out_W3[...]) + out_b3[...])
    yl_o[...] = softmax(_dot(_elu(_dot(he, loc_W1[...]) + loc_b1[...]),
                             loc_W2[...]) + loc_b2[...])
    mu_o[...] = _dot(_elu(_dot(h_full, mu_W1[...]) + mu_b1[...]),
                     mu_W2[...]) + mu_b2[...]
    lv_o[...] = jnp.clip(_dot(_elu(_dot(h_full, lv_W1[...]) + lv_b1[...]),
                              lv_W2[...]) + lv_b2[...], -5.0, 5.0)


def _tc_e(h_ego, h, ae0, ae1, *weights):
    wspecs = [_full(w.shape) for w in weights]
    return pl.pallas_call(
        _tc_e_body,
        grid=(GRID_N,),
        in_specs=[_rows(H), _rows(H), _rows(PW), _rows(PW)] + wspecs,
        out_specs=[_rows(O), _rows(O), _rows(T), _rows(T)],
        out_shape=[
            jax.ShapeDtypeStruct((N, O), jnp.float32),
            jax.ShapeDtypeStruct((N, O), jnp.float32),
            jax.ShapeDtypeStruct((N, T), jnp.float32),
            jax.ShapeDtypeStruct((N, T), jnp.float32),
        ],
    )(h_ego, h, ae0, ae1, *weights)


def kernel(x, edge_index, ego_W1, ego_b1, ego_W2, ego_b2, g1_W, g1_b,
           g2_W, g2_b, ln1_g, ln1_b, ln2_g, ln2_b, exp_W1, exp_b1,
           exp_W2, exp_b2, out_W1, out_b1, out_W2, out_b2, out_W3, out_b3,
           loc_W1, loc_b1, loc_W2, loc_b2, mu_W1, mu_b1, mu_W2, mu_b2,
           lv_W1, lv_b1, lv_W2, lv_b2):
    deg_parts = _sc_deg(edge_index)
    degp = jnp.stack([deg_parts[0, :N, 0], deg_parts[1, :N, 0]], axis=-1)

    h_ego, xw1s, dinv = _tc_a(x, degp, ego_W1, ego_b1, ego_W2, ego_b2, g1_W)

    a1 = _sc_gcn_edges(xw1s, edge_index)
    (xw2s,) = _tc_b(a1[0, :N], a1[1, :N], xw1s, dinv,
                    g1_b, ln1_g, ln1_b, g2_W)

    a2 = _sc_gcn_edges(xw2s, edge_index)
    h, a_proj, b_proj = _tc_c(a2[0, :N], a2[1, :N], xw2s, dinv,
                              g2_b, ln2_g, ln2_b, exp_W1, exp_b1)

    econst = jnp.concatenate(
        [exp_W1[2 * H], exp_W2[:, 0], jnp.full((16,), exp_b2[0])])
    ae = _sc_exposure(h, a_proj, b_proj, econst, edge_index)
    yf, yl, mu, lv = _tc_e(
        h_ego, h, ae[0, :N], ae[1, :N],
        out_W1, out_b1, out_W2, out_b2, out_W3, out_b3,
        loc_W1, loc_b1, loc_W2, loc_b2,
        mu_W1, mu_b1, mu_W2, mu_b2,
        lv_W1, lv_b1, lv_W2, lv_b2)
    return (yf, yl, mu, lv)


# speedup vs baseline: 1.6988x; 1.6988x over previous
import functools

import jax
import jax.numpy as jnp
from jax import lax
from jax.experimental import pallas as pl
from jax.experimental.pallas import tpu as pltpu
from jax.experimental.pallas import tpu_sc as plsc

N = 10000
E = 320000
IN_DIM = 128
H = 32
T = 4
O = 5

NC = 2
NS = 16
NW = NC * NS
NP = 10240
TPN = NP // NS
EW = E // NW
C = 1000
NCH = EW // C

_MESH = plsc.VectorSubcoreMesh(
    core_axis_name="c", subcore_axis_name="s", num_cores=NC, num_subcores=NS)


def _elu(v):
    return jnp.where(v > 0, v, jnp.exp(jnp.minimum(v, 0.0)) - 1.0)


def _ln(v, g, b):
    mu = jnp.mean(v, axis=-1, keepdims=True)
    var = jnp.var(v, axis=-1, keepdims=True)
    return (v - mu) / jnp.sqrt(var + 1e-5) * g + b


def _zero_rows(ref, nrows, width):
    zero = jnp.zeros((16,), jnp.float32)

    def body(i, carry):
        for w0 in range(0, width, 16):
            ref[i, pl.ds(w0, 16)] = zero
        return carry

    lax.fori_loop(0, nrows, body, 0)


@functools.partial(
    pl.kernel,
    out_type=jax.ShapeDtypeStruct((NC, NP, 16), jnp.float32),
    mesh=_MESH,
    compiler_params=pltpu.CompilerParams(use_tc_tiling_on_sc=False, needs_layout_passes=False),
    scratch_types=[
        pltpu.VMEM((C,), jnp.int32),
        pltpu.VMEM((C, 16), jnp.float32),
        pltpu.VMEM_SHARED((NP, 16), jnp.float32),
    ],
)
def _sc_deg(ei_hbm, out_hbm, didx, ones, acc):
    cid = lax.axis_index("c")
    sid = lax.axis_index("s")
    wid = sid * NC + cid

    _zero_rows(ones, TPN, 16)
    pltpu.sync_copy(ones.at[pl.ds(0, TPN)], acc.at[pl.ds(sid * TPN, TPN)])

    one = jnp.ones((16,), jnp.float32)

    def fill(i, carry):
        ones[i, :] = one
        return carry

    lax.fori_loop(0, C, fill, 0)
    plsc.subcore_barrier()

    def chunk(i, carry):
        b = pl.multiple_of(wid * EW + i * C, 8)
        pltpu.sync_copy(ei_hbm.at[1, pl.ds(b, C)], didx)
        pltpu.sync_copy(ones, acc.at[didx], add=True)
        return carry

    lax.fori_loop(0, NCH, chunk, 0)
    plsc.subcore_barrier()
    pltpu.sync_copy(acc.at[pl.ds(sid * TPN, TPN)],
                    out_hbm.at[cid, pl.ds(sid * TPN, TPN)])


@functools.partial(
    pl.kernel,
    out_type=jax.ShapeDtypeStruct((NC, NP, H), jnp.float32),
    mesh=_MESH,
    compiler_params=pltpu.CompilerParams(use_tc_tiling_on_sc=False, needs_layout_passes=False),
    scratch_types=[
        pltpu.VMEM((C,), jnp.int32),
        pltpu.VMEM((C,), jnp.int32),
        pltpu.VMEM((C,), jnp.int32),
        pltpu.VMEM((C,), jnp.int32),
        pltpu.VMEM((C, H), jnp.float32),
        pltpu.VMEM((C, H), jnp.float32),
        pltpu.VMEM_SHARED((NP, H), jnp.float32),
        pltpu.SemaphoreType.DMA,
        pltpu.SemaphoreType.DMA,
    ],
)
def _sc_gcn_edges(xws_hbm, ei_hbm, out_hbm,
                  sidx0, sidx1, didx0, didx1, rows0, rows1, acc,
                  sem0, sem1):
    cid = lax.axis_index("c")
    sid = lax.axis_index("s")
    wid = sid * NC + cid

    _zero_rows(rows0, TPN, H)
    pltpu.sync_copy(rows0.at[pl.ds(0, TPN)], acc.at[pl.ds(sid * TPN, TPN)])
    plsc.subcore_barrier()

    sidx = (sidx0, sidx1)
    didx = (didx0, didx1)
    rows = (rows0, rows1)
    sem = (sem0, sem1)

    def issue(c, b):
        eb = pl.multiple_of(wid * EW + c * C, 8)
        pltpu.sync_copy(ei_hbm.at[0, pl.ds(eb, C)], sidx[b])
        pltpu.sync_copy(ei_hbm.at[1, pl.ds(eb, C)], didx[b])
        return pltpu.async_copy(xws_hbm.at[sidx[b]], rows[b], sem[b])

    cp = [None, None]
    cp[0] = issue(0, 0)
    for c in range(NCH):
        b = c % 2
        nb = (c + 1) % 2
        if c + 1 < NCH:
            cp[nb] = issue(c + 1, nb)
        cp[b].wait()
        pltpu.sync_copy(rows[b], acc.at[didx[b]], add=True)
    plsc.subcore_barrier()
    pltpu.sync_copy(acc.at[pl.ds(sid * TPN, TPN)],
                    out_hbm.at[cid, pl.ds(sid * TPN, TPN)])


PW = 48
CE = 400
NCHE = EW // CE
GPC = CE // 16


@functools.partial(
    pl.kernel,
    out_type=jax.ShapeDtypeStruct((NC, NP, PW), jnp.float32),
    mesh=_MESH,
    compiler_params=pltpu.CompilerParams(use_tc_tiling_on_sc=False, needs_layout_passes=False),
    scratch_types=[
        pltpu.VMEM((CE,), jnp.int32),
        pltpu.VMEM((CE,), jnp.int32),
        pltpu.VMEM((CE, H), jnp.float32),
        pltpu.VMEM((CE, H), jnp.float32),
        pltpu.VMEM((CE, 16), jnp.float32),
        pltpu.VMEM((CE, 16), jnp.float32),
        pltpu.VMEM((CE, 17), jnp.float32),
        pltpu.VMEM((CE, 17), jnp.float32),
        pltpu.VMEM((CE, PW), jnp.float32),
        pltpu.VMEM((48,), jnp.float32),
        pltpu.VMEM_SHARED((NP, PW), jnp.float32),
        pltpu.SemaphoreType.DMA,
        pltpu.SemaphoreType.DMA,
        pltpu.SemaphoreType.DMA,
        pltpu.SemaphoreType.DMA,
    ],
)
def _sc_exposure(h_hbm, a_hbm, b_hbm, econst_hbm, ei_hbm, out_hbm,
                 sidx, didx, hs, hd, ad, bs, ad17, bs17,
                 outb, econ, acc, sem_s, sem_d, sem_a, sem_b):
    cid = lax.axis_index("c")
    sid = lax.axis_index("s")
    wid = sid * NC + cid

    _zero_rows(outb, CE, PW)
    pltpu.sync_copy(outb.at[pl.ds(0, 320)], acc.at[pl.ds(sid * TPN, 320)])
    pltpu.sync_copy(outb.at[pl.ds(0, 320)],
                    acc.at[pl.ds(sid * TPN + 320, 320)])
    pltpu.sync_copy(econst_hbm, econ)
    plsc.subcore_barrier()

    w1v = econ[pl.ds(0, 16)]
    w2v = econ[pl.ds(16, 16)]
    b2v = econ[pl.ds(32, 16)]

    iota16 = lax.broadcasted_iota(jnp.int32, (16,), 0)

    def cvec(k):
        return jnp.full((16,), k, jnp.int32)

    _gdn = lax.GatherDimensionNumbers(
        offset_dims=(), collapsed_slice_dims=(0,), start_index_map=(0,))

    def vsum16(v):
        for sh in (8, 4, 2, 1):
            perm = jnp.bitwise_xor(iota16, sh)
            v = v + lax.gather(v, perm[:, None], dimension_numbers=_gdn,
                               slice_sizes=(1,),
                               mode=lax.GatherScatterMode.PROMISE_IN_BOUNDS)
        return v

    def chunk(i, carry):
        eb = pl.multiple_of(wid * EW + i * CE, 8)
        pltpu.sync_copy(ei_hbm.at[0, pl.ds(eb, CE)], sidx)
        pltpu.sync_copy(ei_hbm.at[1, pl.ds(eb, CE)], didx)
        cp_s = pltpu.async_copy(h_hbm.at[sidx], hs, sem_s)
        cp_d = pltpu.async_copy(h_hbm.at[didx], hd, sem_d)
        cp_a = pltpu.async_copy(a_hbm.at[didx], ad, sem_a)
        cp_b = pltpu.async_copy(b_hbm.at[sidx], bs, sem_b)
        cp_s.wait()
        cp_d.wait()
        cp_a.wait()
        cp_b.wait()

        def restage(r4, carry2):
            for j in range(4):
                r = r4 * 4 + j
                ad17[r, pl.ds(0, 16)] = ad[r, :]
                bs17[r, pl.ds(0, 16)] = bs[r, :]
            return carry2

        lax.fori_loop(0, CE // 4, restage, 0)

        def group(g, carry2):
            rows = g * 16 + iota16
            s = jnp.zeros((16,), jnp.float32)
            for e in range(16):
                r = g * 16 + e
                d0 = hd[r, pl.ds(0, 16)] - hs[r, pl.ds(0, 16)]
                d1 = hd[r, pl.ds(16, 16)] - hs[r, pl.ds(16, 16)]
                se = vsum16(d0 * d0 + d1 * d1)
                s = s + jnp.where(iota16 == e, se, 0.0)
            sim = jnp.exp(s * (-1.0 / (2.0 * H)))
            t = jnp.zeros((16,), jnp.float32)
            for k in range(16):
                ck = cvec(k)
                zk = (plsc.load_gather(ad17, [rows, ck])
                      + plsc.load_gather(bs17, [rows, ck])
                      + sim * w1v[k])
                uk = jnp.where(zk > 0, zk,
                               jnp.exp(jnp.minimum(zk, 0.0)) - 1.0)
                t = t + uk * w2v[k]
            t = t + b2v[0]
            wgt = 1.0 / (1.0 + jnp.exp(-t))
            for e in range(16):
                r = g * 16 + e
                we = wgt[e]
                outb[r, pl.ds(0, 16)] = hs[r, pl.ds(0, 16)] * we
                outb[r, pl.ds(16, 16)] = hs[r, pl.ds(16, 16)] * we
            plsc.store_scatter(outb, [rows, cvec(H)], wgt)
            return carry2

        lax.fori_loop(0, GPC, group, 0)
        pltpu.sync_copy(outb, acc.at[didx], add=True)
        return carry

    lax.fori_loop(0, NCHE, chunk, 0)
    plsc.subcore_barrier()
    pltpu.sync_copy(acc.at[pl.ds(sid * TPN, TPN)],
                    out_hbm.at[cid, pl.ds(sid * TPN, TPN)])


RB = 2000
GRID_N = N // RB
EB = 8000
GRID_E = E // EB


def _full(shape):
    return pl.BlockSpec(shape, lambda i: tuple(0 for _ in shape))


def _rows(width):
    return pl.BlockSpec((RB, width), lambda i: (i, 0))


def _dot(a, b):
    return jnp.dot(a, b, preferred_element_type=jnp.float32)


def _tc_a_body(x, dp0, dp1, ego_W1, ego_b1, ego_W2, ego_b2, g1_W,
               h_ego_o, xw1s_o, dinv_o):
    xb = x[...]
    h_ego_o[...] = _dot(_elu(_dot(xb, ego_W1[...]) + ego_b1[...]),
                        ego_W2[...]) + ego_b2[...]
    deg = 1.0 + dp0[0][:, 0:1] + dp1[0][:, 0:1]
    dinv = lax.rsqrt(deg)
    dinv_o[...] = dinv
    xw1s_o[...] = _dot(xb, g1_W[...]) * dinv


def _part(width, c):
    return pl.BlockSpec((1, RB, width), lambda i, c=c: (c, i, 0))


def _tc_a(x, degp, ego_W1, ego_b1, ego_W2, ego_b2, g1_W):
    return pl.pallas_call(
        _tc_a_body,
        grid=(GRID_N,),
        in_specs=[
            _rows(IN_DIM), _part(16, 0), _part(16, 1),
            _full((IN_DIM, H)), _full((H,)), _full((H, H)), _full((H,)),
            _full((IN_DIM, H)),
        ],
        out_specs=[_rows(H), _rows(H), _rows(1)],
        out_shape=[
            jax.ShapeDtypeStruct((N, H), jnp.float32),
            jax.ShapeDtypeStruct((N, H), jnp.float32),
            jax.ShapeDtypeStruct((N, 1), jnp.float32),
        ],
    )(x, degp, degp, ego_W1, ego_b1, ego_W2, ego_b2, g1_W)


def _tc_b_body(a0, a1, xws, dinv, gb, lng, lnb, W2, xw2s_o):
    dv = dinv[...]
    g1out = dv * (a0[0] + a1[0] + xws[...]) + gb[...]
    h1 = _elu(_ln(g1out, lng[...], lnb[...]))
    xw2s_o[...] = _dot(h1, W2[...]) * dv


def _tc_b(a0, a1, xws, dinv, gb, lng, lnb, W2):
    return pl.pallas_call(
        _tc_b_body,
        grid=(GRID_N,),
        in_specs=[
            _part(H, 0), _part(H, 1), _rows(H), _rows(1),
            _full((H,)), _full((H,)), _full((H,)), _full((H, H)),
        ],
        out_specs=[_rows(H)],
        out_shape=[jax.ShapeDtypeStruct((N, H), jnp.float32)],
    )(a0, a1, xws, dinv, gb, lng, lnb, W2)


def _tc_c_body(a0, a1, xws, dinv, gb, lng, lnb, eW1, eb1, h_o, a_o, b_o):
    g2out = dinv[...] * (a0[0] + a1[0] + xws[...]) + gb[...]
    h = _elu(_ln(g2out, lng[...], lnb[...]))
    h_o[...] = h
    w1 = eW1[...]
    a_o[...] = _dot(h, w1[0:H]) + eb1[...]
    b_o[...] = _dot(h, w1[H:2 * H])


def _tc_c(a0, a1, xws, dinv, gb, lng, lnb, eW1, eb1):
    return pl.pallas_call(
        _tc_c_body,
        grid=(GRID_N,),
        in_specs=[
            _part(H, 0), _part(H, 1), _rows(H), _rows(1),
            _full((H,)), _full((H,)), _full((H,)),
            _full((2 * H + 1, 16)), _full((16,)),
        ],
        out_specs=[_rows(H), _rows(16), _rows(16)],
        out_shape=[
            jax.ShapeDtypeStruct((N, H), jnp.float32),
            jax.ShapeDtypeStruct((N, 16), jnp.float32),
            jax.ShapeDtypeStruct((N, 16), jnp.float32),
        ],
    )(a0, a1, xws, dinv, gb, lng, lnb, eW1, eb1)


def _tc_e_body(h_ego, h, ae0, ae1,
               out_W1, out_b1, out_W2, out_b2, out_W3, out_b3,
               loc_W1, loc_b1, loc_W2, loc_b2,
               mu_W1, mu_b1, mu_W2, mu_b2,
               lv_W1, lv_b1, lv_W2, lv_b2,
               yf_o, yl_o, mu_o, lv_o):
    he = h_ego[...]
    hb = h[...]
    a = ae0[0] + ae1[0]
    h_exp = a[:, 0:H] / jnp.maximum(a[:, H:H + 1], 1e-8)
    h_full = jnp.concatenate([he, hb, h_exp], axis=-1)

    def softmax(v):
        m = jnp.max(v, axis=-1, keepdims=True)
        e = jnp.exp(v - m)
        return e / jnp.sum(e, axis=-1, keepdims=True)

    o = _elu(_dot(h_full, out_W1[...]) + out_b1[...])
    o = _elu(_dot(o, out_W2[...]) + out_b2[...])
    yf_o[...] = softmax(_dot(o, out_W3[...]) + out_b3[...])
    yl_o[...] = softmax(_dot(_elu(_dot(he, loc_W1[...]) + loc_b1[...]),
                             loc_W2[...]) + loc_b2[...])
    mu_o[...] = _dot(_elu(_dot(h_full, mu_W1[...]) + mu_b1[...]),
                     mu_W2[...]) + mu_b2[...]
    lv_o[...] = jnp.clip(_dot(_elu(_dot(h_full, lv_W1[...]) + lv_b1[...]),
                              lv_W2[...]) + lv_b2[...], -5.0, 5.0)


def _tc_e(h_ego, h, ae0, ae1, *weights):
    wspecs = [_full(w.shape) for w in weights]
    return pl.pallas_call(
        _tc_e_body,
        grid=(GRID_N,),
        in_specs=[_rows(H), _rows(H), _part(PW, 0), _part(PW, 1)] + wspecs,
        out_specs=[_rows(O), _rows(O), _rows(T), _rows(T)],
        out_shape=[
            jax.ShapeDtypeStruct((N, O), jnp.float32),
            jax.ShapeDtypeStruct((N, O), jnp.float32),
            jax.ShapeDtypeStruct((N, T), jnp.float32),
            jax.ShapeDtypeStruct((N, T), jnp.float32),
        ],
    )(h_ego, h, ae0, ae1, *weights)


def kernel(x, edge_index, ego_W1, ego_b1, ego_W2, ego_b2, g1_W, g1_b,
           g2_W, g2_b, ln1_g, ln1_b, ln2_g, ln2_b, exp_W1, exp_b1,
           exp_W2, exp_b2, out_W1, out_b1, out_W2, out_b2, out_W3, out_b3,
           loc_W1, loc_b1, loc_W2, loc_b2, mu_W1, mu_b1, mu_W2, mu_b2,
           lv_W1, lv_b1, lv_W2, lv_b2):
    deg_parts = _sc_deg(edge_index)

    h_ego, xw1s, dinv = _tc_a(x, deg_parts, ego_W1, ego_b1, ego_W2,
                              ego_b2, g1_W)

    a1 = _sc_gcn_edges(xw1s, edge_index)
    (xw2s,) = _tc_b(a1, a1, xw1s, dinv, g1_b, ln1_g, ln1_b, g2_W)

    a2 = _sc_gcn_edges(xw2s, edge_index)
    h, a_proj, b_proj = _tc_c(a2, a2, xw2s, dinv,
                              g2_b, ln2_g, ln2_b, exp_W1, exp_b1)

    econst = jnp.concatenate(
        [exp_W1[2 * H], exp_W2[:, 0], jnp.full((16,), exp_b2[0])])
    ae = _sc_exposure(h, a_proj, b_proj, econst, edge_index)
    yf, yl, mu, lv = _tc_e(
        h_ego, h, ae, ae,
        out_W1, out_b1, out_W2, out_b2, out_W3, out_b3,
        loc_W1, loc_b1, loc_W2, loc_b2,
        mu_W1, mu_b1, mu_W2, mu_b2,
        lv_W1, lv_b1, lv_W2, lv_b2)
    return (yf, yl, mu, lv)


# speedup vs baseline: 1.7950x; 1.0566x over previous
import functools

import jax
import jax.numpy as jnp
from jax import lax
from jax.experimental import pallas as pl
from jax.experimental.pallas import tpu as pltpu
from jax.experimental.pallas import tpu_sc as plsc

N = 10000
E = 320000
IN_DIM = 128
H = 32
T = 4
O = 5

NC = 2
NS = 16
NW = NC * NS
NP = 10240
TPN = NP // NS
EW = E // NW
C = 1000
NCH = EW // C

_MESH = plsc.VectorSubcoreMesh(
    core_axis_name="c", subcore_axis_name="s", num_cores=NC, num_subcores=NS)


def _elu(v):
    return jnp.where(v > 0, v, jnp.exp(jnp.minimum(v, 0.0)) - 1.0)


def _ln(v, g, b):
    mu = jnp.mean(v, axis=-1, keepdims=True)
    var = jnp.var(v, axis=-1, keepdims=True)
    return (v - mu) / jnp.sqrt(var + 1e-5) * g + b


def _zero_rows(ref, nrows, width):
    zero = jnp.zeros((16,), jnp.float32)

    def body(i, carry):
        for w0 in range(0, width, 16):
            ref[i, pl.ds(w0, 16)] = zero
        return carry

    lax.fori_loop(0, nrows, body, 0)


@functools.partial(
    pl.kernel,
    out_type=jax.ShapeDtypeStruct((NC, NP, 16), jnp.float32),
    mesh=_MESH,
    compiler_params=pltpu.CompilerParams(use_tc_tiling_on_sc=False, needs_layout_passes=False),
    scratch_types=[
        pltpu.VMEM((C,), jnp.int32),
        pltpu.VMEM((C, 16), jnp.float32),
        pltpu.VMEM_SHARED((NP, 16), jnp.float32),
    ],
)
def _sc_deg(ei_hbm, out_hbm, didx, ones, acc):
    cid = lax.axis_index("c")
    sid = lax.axis_index("s")
    wid = sid * NC + cid

    _zero_rows(ones, TPN, 16)
    pltpu.sync_copy(ones.at[pl.ds(0, TPN)], acc.at[pl.ds(sid * TPN, TPN)])

    one = jnp.ones((16,), jnp.float32)

    def fill(i, carry):
        ones[i, :] = one
        return carry

    lax.fori_loop(0, C, fill, 0)
    plsc.subcore_barrier()

    def chunk(i, carry):
        b = pl.multiple_of(wid * EW + i * C, 8)
        pltpu.sync_copy(ei_hbm.at[1, pl.ds(b, C)], didx)
        pltpu.sync_copy(ones, acc.at[didx], add=True)
        return carry

    lax.fori_loop(0, NCH, chunk, 0)
    plsc.subcore_barrier()
    pltpu.sync_copy(acc.at[pl.ds(sid * TPN, TPN)],
                    out_hbm.at[cid, pl.ds(sid * TPN, TPN)])


@functools.partial(
    pl.kernel,
    out_type=jax.ShapeDtypeStruct((NC, NP, H), jnp.float32),
    mesh=_MESH,
    compiler_params=pltpu.CompilerParams(use_tc_tiling_on_sc=False, needs_layout_passes=False),
    scratch_types=[
        pltpu.VMEM((C,), jnp.int32),
        pltpu.VMEM((C,), jnp.int32),
        pltpu.VMEM((C,), jnp.int32),
        pltpu.VMEM((C,), jnp.int32),
        pltpu.VMEM((C, H), jnp.float32),
        pltpu.VMEM((C, H), jnp.float32),
        pltpu.VMEM_SHARED((NP, H), jnp.float32),
        pltpu.SemaphoreType.DMA,
        pltpu.SemaphoreType.DMA,
    ],
)
def _sc_gcn_edges(xws_hbm, ei_hbm, out_hbm,
                  sidx0, sidx1, didx0, didx1, rows0, rows1, acc,
                  sem0, sem1):
    cid = lax.axis_index("c")
    sid = lax.axis_index("s")
    wid = sid * NC + cid

    _zero_rows(rows0, TPN, H)
    pltpu.sync_copy(rows0.at[pl.ds(0, TPN)], acc.at[pl.ds(sid * TPN, TPN)])
    plsc.subcore_barrier()

    sidx = (sidx0, sidx1)
    didx = (didx0, didx1)
    rows = (rows0, rows1)
    sem = (sem0, sem1)

    def issue(c, b):
        eb = pl.multiple_of(wid * EW + c * C, 8)
        pltpu.sync_copy(ei_hbm.at[0, pl.ds(eb, C)], sidx[b])
        pltpu.sync_copy(ei_hbm.at[1, pl.ds(eb, C)], didx[b])
        return pltpu.async_copy(xws_hbm.at[sidx[b]], rows[b], sem[b])

    cp = [None, None]
    cp[0] = issue(0, 0)
    for c in range(NCH):
        b = c % 2
        nb = (c + 1) % 2
        if c + 1 < NCH:
            cp[nb] = issue(c + 1, nb)
        cp[b].wait()
        pltpu.sync_copy(rows[b], acc.at[didx[b]], add=True)
    plsc.subcore_barrier()
    pltpu.sync_copy(acc.at[pl.ds(sid * TPN, TPN)],
                    out_hbm.at[cid, pl.ds(sid * TPN, TPN)])


PW = 48
CE = 400
NCHE = EW // CE
GPC = CE // 16


HF = CE // 2


@functools.partial(
    pl.kernel,
    out_type=jax.ShapeDtypeStruct((NC, NP, PW), jnp.float32),
    mesh=_MESH,
    compiler_params=pltpu.CompilerParams(use_tc_tiling_on_sc=False, needs_layout_passes=False),
    scratch_types=[
        pltpu.VMEM((2, HF), jnp.int32),
        pltpu.VMEM((2, HF), jnp.int32),
        pltpu.VMEM((2, HF, H), jnp.float32),
        pltpu.VMEM((2, HF, H), jnp.float32),
        pltpu.VMEM((2, HF, 16), jnp.float32),
        pltpu.VMEM((2, HF, 16), jnp.float32),
        pltpu.VMEM((2, HF, 17), jnp.float32),
        pltpu.VMEM((2, HF, 17), jnp.float32),
        pltpu.VMEM((2, HF, PW), jnp.float32),
        pltpu.VMEM((48,), jnp.float32),
        pltpu.VMEM_SHARED((NP, PW), jnp.float32),
        pltpu.SemaphoreType.DMA,
        pltpu.SemaphoreType.DMA,
        pltpu.SemaphoreType.DMA,
        pltpu.SemaphoreType.DMA,
        pltpu.SemaphoreType.DMA,
        pltpu.SemaphoreType.DMA,
        pltpu.SemaphoreType.DMA,
        pltpu.SemaphoreType.DMA,
    ],
)
def _sc_exposure(h_hbm, a_hbm, b_hbm, econst_hbm, ei_hbm, out_hbm,
                 sidx, didx, hs, hd, ad, bs, ad17, bs17,
                 outb, econ, acc,
                 ss0, sd0, sa0, sb0, ss1, sd1, sa1, sb1):
    cid = lax.axis_index("c")
    sid = lax.axis_index("s")
    wid = sid * NC + cid

    _zero_rows(outb.at[0], HF, PW)
    pltpu.sync_copy(outb.at[0], acc.at[pl.ds(sid * TPN, HF)])
    pltpu.sync_copy(outb.at[0], acc.at[pl.ds(sid * TPN + HF, HF)])
    pltpu.sync_copy(outb.at[0], acc.at[pl.ds(sid * TPN + 2 * HF, HF)])
    pltpu.sync_copy(outb.at[0].at[pl.ds(0, TPN - 3 * HF)],
                    acc.at[pl.ds(sid * TPN + 3 * HF, TPN - 3 * HF)])
    pltpu.sync_copy(econst_hbm, econ)
    plsc.subcore_barrier()

    w1v = econ[pl.ds(0, 16)]
    w2v = econ[pl.ds(16, 16)]
    b2v = econ[pl.ds(32, 16)]

    iota16 = lax.broadcasted_iota(jnp.int32, (16,), 0)

    def cvec(k):
        return jnp.full((16,), k, jnp.int32)

    _gdn = lax.GatherDimensionNumbers(
        offset_dims=(), collapsed_slice_dims=(0,), start_index_map=(0,))

    def vsum16(v):
        for sh in (8, 4, 2, 1):
            perm = jnp.bitwise_xor(iota16, sh)
            v = v + lax.gather(v, perm[:, None], dimension_numbers=_gdn,
                               slice_sizes=(1,),
                               mode=lax.GatherScatterMode.PROMISE_IN_BOUNDS)
        return v

    sems = ((ss0, sd0, sa0, sb0), (ss1, sd1, sa1, sb1))

    def issue(base, hf):
        b = pl.multiple_of(base, 8)
        pltpu.sync_copy(ei_hbm.at[0, pl.ds(b, HF)], sidx.at[hf])
        pltpu.sync_copy(ei_hbm.at[1, pl.ds(b, HF)], didx.at[hf])
        s = sems[hf]
        pltpu.async_copy(h_hbm.at[sidx.at[hf]], hs.at[hf], s[0])
        pltpu.async_copy(h_hbm.at[didx.at[hf]], hd.at[hf], s[1])
        pltpu.async_copy(a_hbm.at[didx.at[hf]], ad.at[hf], s[2])
        pltpu.async_copy(b_hbm.at[sidx.at[hf]], bs.at[hf], s[3])

    def wait(hf):
        s = sems[hf]
        pltpu.make_async_copy(h_hbm.at[sidx.at[hf]], hs.at[hf], s[0]).wait()
        pltpu.make_async_copy(h_hbm.at[didx.at[hf]], hd.at[hf], s[1]).wait()
        pltpu.make_async_copy(a_hbm.at[didx.at[hf]], ad.at[hf], s[2]).wait()
        pltpu.make_async_copy(b_hbm.at[sidx.at[hf]], bs.at[hf], s[3]).wait()

    def compute(hf):
        def restage(r4, carry2):
            for j in range(4):
                r = r4 * 4 + j
                ad17[hf, r, pl.ds(0, 16)] = ad[hf, r, :]
                bs17[hf, r, pl.ds(0, 16)] = bs[hf, r, :]
            return carry2

        lax.fori_loop(0, HF // 4, restage, 0)

        def group(g, carry2):
            rows = g * 16 + iota16
            s = jnp.zeros((16,), jnp.float32)
            for e in range(16):
                r = g * 16 + e
                d0 = hd[hf, r, pl.ds(0, 16)] - hs[hf, r, pl.ds(0, 16)]
                d1 = hd[hf, r, pl.ds(16, 16)] - hs[hf, r, pl.ds(16, 16)]
                se = vsum16(d0 * d0 + d1 * d1)
                s = s + jnp.where(iota16 == e, se, 0.0)
            sim = jnp.exp(s * (-1.0 / (2.0 * H)))
            t = jnp.zeros((16,), jnp.float32)
            for k in range(16):
                ck = cvec(k)
                zk = (plsc.load_gather(ad17.at[hf], [rows, ck])
                      + plsc.load_gather(bs17.at[hf], [rows, ck])
                      + sim * w1v[k])
                uk = jnp.where(zk > 0, zk,
                               jnp.exp(jnp.minimum(zk, 0.0)) - 1.0)
                t = t + uk * w2v[k]
            t = t + b2v[0]
            wgt = 1.0 / (1.0 + jnp.exp(-t))
            for e in range(16):
                r = g * 16 + e
                we = wgt[e]
                outb[hf, r, pl.ds(0, 16)] = hs[hf, r, pl.ds(0, 16)] * we
                outb[hf, r, pl.ds(16, 16)] = hs[hf, r, pl.ds(16, 16)] * we
            plsc.store_scatter(outb.at[hf], [rows, cvec(H)], wgt)
            return carry2

        lax.fori_loop(0, HF // 16, group, 0)
        pltpu.sync_copy(outb.at[hf], acc.at[didx.at[hf]], add=True)

    issue(wid * EW, 0)

    def chunk(i, carry):
        eb = wid * EW + i * CE
        wait(0)
        issue(eb + HF, 1)
        compute(0)
        wait(1)

        @pl.when(i < NCHE - 1)
        def _():
            issue(eb + CE, 0)

        compute(1)
        return carry

    lax.fori_loop(0, NCHE, chunk, 0)
    plsc.subcore_barrier()
    pltpu.sync_copy(acc.at[pl.ds(sid * TPN, TPN)],
                    out_hbm.at[cid, pl.ds(sid * TPN, TPN)])


RB = 2000
GRID_N = N // RB
EB = 8000
GRID_E = E // EB


def _full(shape):
    return pl.BlockSpec(shape, lambda i: tuple(0 for _ in shape))


def _rows(width):
    return pl.BlockSpec((RB, width), lambda i: (i, 0))


def _dot(a, b):
    return jnp.dot(a, b, preferred_element_type=jnp.float32)


def _tc_a_body(x, dp0, dp1, ego_W1, ego_b1, ego_W2, ego_b2, g1_W,
               h_ego_o, xw1s_o, dinv_o):
    xb = x[...]
    h_ego_o[...] = _dot(_elu(_dot(xb, ego_W1[...]) + ego_b1[...]),
                        ego_W2[...]) + ego_b2[...]
    deg = 1.0 + dp0[0][:, 0:1] + dp1[0][:, 0:1]
    dinv = lax.rsqrt(deg)
    dinv_o[...] = dinv
    xw1s_o[...] = _dot(xb, g1_W[...]) * dinv


def _part(width, c):
    return pl.BlockSpec((1, RB, width), lambda i, c=c: (c, i, 0))


def _tc_a(x, degp, ego_W1, ego_b1, ego_W2, ego_b2, g1_W):
    return pl.pallas_call(
        _tc_a_body,
        grid=(GRID_N,),
        in_specs=[
            _rows(IN_DIM), _part(16, 0), _part(16, 1),
            _full((IN_DIM, H)), _full((H,)), _full((H, H)), _full((H,)),
            _full((IN_DIM, H)),
        ],
        out_specs=[_rows(H), _rows(H), _rows(1)],
        out_shape=[
            jax.ShapeDtypeStruct((N, H), jnp.float32),
            jax.ShapeDtypeStruct((N, H), jnp.float32),
            jax.ShapeDtypeStruct((N, 1), jnp.float32),
        ],
    )(x, degp, degp, ego_W1, ego_b1, ego_W2, ego_b2, g1_W)


def _tc_b_body(a0, a1, xws, dinv, gb, lng, lnb, W2, xw2s_o):
    dv = dinv[...]
    g1out = dv * (a0[0] + a1[0] + xws[...]) + gb[...]
    h1 = _elu(_ln(g1out, lng[...], lnb[...]))
    xw2s_o[...] = _dot(h1, W2[...]) * dv


def _tc_b(a0, a1, xws, dinv, gb, lng, lnb, W2):
    return pl.pallas_call(
        _tc_b_body,
        grid=(GRID_N,),
        in_specs=[
            _part(H, 0), _part(H, 1), _rows(H), _rows(1),
            _full((H,)), _full((H,)), _full((H,)), _full((H, H)),
        ],
        out_specs=[_rows(H)],
        out_shape=[jax.ShapeDtypeStruct((N, H), jnp.float32)],
    )(a0, a1, xws, dinv, gb, lng, lnb, W2)


def _tc_c_body(a0, a1, xws, dinv, gb, lng, lnb, eW1, eb1, h_o, a_o, b_o):
    g2out = dinv[...] * (a0[0] + a1[0] + xws[...]) + gb[...]
    h = _elu(_ln(g2out, lng[...], lnb[...]))
    h_o[...] = h
    w1 = eW1[...]
    a_o[...] = _dot(h, w1[0:H]) + eb1[...]
    b_o[...] = _dot(h, w1[H:2 * H])


def _tc_c(a0, a1, xws, dinv, gb, lng, lnb, eW1, eb1):
    return pl.pallas_call(
        _tc_c_body,
        grid=(GRID_N,),
        in_specs=[
            _part(H, 0), _part(H, 1), _rows(H), _rows(1),
            _full((H,)), _full((H,)), _full((H,)),
            _full((2 * H + 1, 16)), _full((16,)),
        ],
        out_specs=[_rows(H), _rows(16), _rows(16)],
        out_shape=[
            jax.ShapeDtypeStruct((N, H), jnp.float32),
            jax.ShapeDtypeStruct((N, 16), jnp.float32),
            jax.ShapeDtypeStruct((N, 16), jnp.float32),
        ],
    )(a0, a1, xws, dinv, gb, lng, lnb, eW1, eb1)


def _tc_e_body(h_ego, h, ae0, ae1,
               out_W1, out_b1, out_W2, out_b2, out_W3, out_b3,
               loc_W1, loc_b1, loc_W2, loc_b2,
               mu_W1, mu_b1, mu_W2, mu_b2,
               lv_W1, lv_b1, lv_W2, lv_b2,
               yf_o, yl_o, mu_o, lv_o):
    he = h_ego[...]
    hb = h[...]
    a = ae0[0] + ae1[0]
    h_exp = a[:, 0:H] / jnp.maximum(a[:, H:H + 1], 1e-8)
    h_full = jnp.concatenate([he, hb, h_exp], axis=-1)

    def softmax(v):
        m = jnp.max(v, axis=-1, keepdims=True)
        e = jnp.exp(v - m)
        return e / jnp.sum(e, axis=-1, keepdims=True)

    o = _elu(_dot(h_full, out_W1[...]) + out_b1[...])
    o = _elu(_dot(o, out_W2[...]) + out_b2[...])
    yf_o[...] = softmax(_dot(o, out_W3[...]) + out_b3[...])
    yl_o[...] = softmax(_dot(_elu(_dot(he, loc_W1[...]) + loc_b1[...]),
                             loc_W2[...]) + loc_b2[...])
    mu_o[...] = _dot(_elu(_dot(h_full, mu_W1[...]) + mu_b1[...]),
                     mu_W2[...]) + mu_b2[...]
    lv_o[...] = jnp.clip(_dot(_elu(_dot(h_full, lv_W1[...]) + lv_b1[...]),
                              lv_W2[...]) + lv_b2[...], -5.0, 5.0)


def _tc_e(h_ego, h, ae0, ae1, *weights):
    wspecs = [_full(w.shape) for w in weights]
    return pl.pallas_call(
        _tc_e_body,
        grid=(GRID_N,),
        in_specs=[_rows(H), _rows(H), _part(PW, 0), _part(PW, 1)] + wspecs,
        out_specs=[_rows(O), _rows(O), _rows(T), _rows(T)],
        out_shape=[
            jax.ShapeDtypeStruct((N, O), jnp.float32),
            jax.ShapeDtypeStruct((N, O), jnp.float32),
            jax.ShapeDtypeStruct((N, T), jnp.float32),
            jax.ShapeDtypeStruct((N, T), jnp.float32),
        ],
    )(h_ego, h, ae0, ae1, *weights)


def kernel(x, edge_index, ego_W1, ego_b1, ego_W2, ego_b2, g1_W, g1_b,
           g2_W, g2_b, ln1_g, ln1_b, ln2_g, ln2_b, exp_W1, exp_b1,
           exp_W2, exp_b2, out_W1, out_b1, out_W2, out_b2, out_W3, out_b3,
           loc_W1, loc_b1, loc_W2, loc_b2, mu_W1, mu_b1, mu_W2, mu_b2,
           lv_W1, lv_b1, lv_W2, lv_b2):
    deg_parts = _sc_deg(edge_index)

    h_ego, xw1s, dinv = _tc_a(x, deg_parts, ego_W1, ego_b1, ego_W2,
                              ego_b2, g1_W)

    a1 = _sc_gcn_edges(xw1s, edge_index)
    (xw2s,) = _tc_b(a1, a1, xw1s, dinv, g1_b, ln1_g, ln1_b, g2_W)

    a2 = _sc_gcn_edges(xw2s, edge_index)
    h, a_proj, b_proj = _tc_c(a2, a2, xw2s, dinv,
                              g2_b, ln2_g, ln2_b, exp_W1, exp_b1)

    econst = jnp.concatenate(
        [exp_W1[2 * H], exp_W2[:, 0], jnp.full((16,), exp_b2[0])])
    ae = _sc_exposure(h, a_proj, b_proj, econst, edge_index)
    yf, yl, mu, lv = _tc_e(
        h_ego, h, ae, ae,
        out_W1, out_b1, out_W2, out_b2, out_W3, out_b3,
        loc_W1, loc_b1, loc_W2, loc_b2,
        mu_W1, mu_b1, mu_W2, mu_b2,
        lv_W1, lv_b1, lv_W2, lv_b2)
    return (yf, yl, mu, lv)
